# baseline (speedup 1.0000x reference)
import functools

import jax
import jax.numpy as jnp
from jax import lax
from jax.experimental import pallas as pl
from jax.experimental.pallas import tpu as pltpu
from jax.experimental.pallas import tpu_sc as plsc

H = 128
NC = 2
NS = 16
NW = NC * NS
CHUNK = 128


def _pick_bm(n, cap=2048):
    for bm in (2048, 1024, 1000, 512, 500, 256, 250, 200, 128, 125, 100, 64,
               50, 40, 32, 25, 16, 10, 8, 5, 4, 2, 1):
        if bm <= cap and n % bm == 0:
            return bm
    return 1


def _ln(h, g, be):
    mu = jnp.mean(h, axis=-1, keepdims=True)
    var = jnp.mean((h - mu) ** 2, axis=-1, keepdims=True)
    return (h - mu) * lax.rsqrt(var + 1e-5) * g + be


def _matmul_bias(x, w, b):
    n, k = x.shape
    m = w.shape[1]
    bm = _pick_bm(n)

    def body(x_ref, w_ref, b_ref, o_ref):
        o_ref[...] = jnp.dot(x_ref[...], w_ref[...],
                             preferred_element_type=jnp.float32) + b_ref[...]

    return pl.pallas_call(
        body,
        grid=(n // bm,),
        in_specs=[pl.BlockSpec((bm, k), lambda i: (i, 0)),
                  pl.BlockSpec((k, m), lambda i: (0, 0)),
                  pl.BlockSpec((1, m), lambda i: (0, 0))],
        out_specs=pl.BlockSpec((bm, m), lambda i: (i, 0)),
        out_shape=jax.ShapeDtypeStruct((n, m), jnp.float32),
    )(x, w, b.reshape(1, m))


def _mlp(x, p, residual=False, norm=True):
    n, k = x.shape
    m = p['w2'].shape[1]
    bm = _pick_bm(n)
    g = p['g'].reshape(1, m) if norm else jnp.zeros((1, m), jnp.float32)
    be = p['be'].reshape(1, m) if norm else jnp.zeros((1, m), jnp.float32)

    def body(x_ref, w1_ref, b1_ref, w2_ref, b2_ref, g_ref, be_ref, o_ref):
        h = jax.nn.silu(jnp.dot(x_ref[...], w1_ref[...],
                                preferred_element_type=jnp.float32)
                        + b1_ref[...])
        h = jnp.dot(h, w2_ref[...], preferred_element_type=jnp.float32) \
            + b2_ref[...]
        if norm:
            h = _ln(h, g_ref[...], be_ref[...])
        if residual:
            h = h + x_ref[...]
        o_ref[...] = h

    hdim = p['w1'].shape[1]
    return pl.pallas_call(
        body,
        grid=(n // bm,),
        in_specs=[pl.BlockSpec((bm, k), lambda i: (i, 0)),
                  pl.BlockSpec((k, hdim), lambda i: (0, 0)),
                  pl.BlockSpec((1, hdim), lambda i: (0, 0)),
                  pl.BlockSpec((hdim, m), lambda i: (0, 0)),
                  pl.BlockSpec((1, m), lambda i: (0, 0)),
                  pl.BlockSpec((1, m), lambda i: (0, 0)),
                  pl.BlockSpec((1, m), lambda i: (0, 0))],
        out_specs=pl.BlockSpec((bm, m), lambda i: (i, 0)),
        out_shape=jax.ShapeDtypeStruct((n, m), jnp.float32),
    )(x, p['w1'], p['b1'].reshape(1, hdim), p['w2'], p['b2'].reshape(1, m),
      g, be)


def _edge_update(g1, g2, pe, ef, p):
    n = g1.shape[0]
    bm = _pick_bm(n)

    def body(g1_ref, g2_ref, pe_ref, ef_ref, w2_ref, b2_ref, g_ref, be_ref,
             o_ref):
        h = jax.nn.silu(g1_ref[...] + g2_ref[...] + pe_ref[...])
        h = jnp.dot(h, w2_ref[...], preferred_element_type=jnp.float32) \
            + b2_ref[...]
        o_ref[...] = ef_ref[...] + _ln(h, g_ref[...], be_ref[...])

    full = lambda i: (0, 0)
    row = lambda i: (i, 0)
    return pl.pallas_call(
        body,
        grid=(n // bm,),
        in_specs=[pl.BlockSpec((bm, H), row)] * 4 +
                 [pl.BlockSpec((H, H), full),
                  pl.BlockSpec((1, H), full),
                  pl.BlockSpec((1, H), full),
                  pl.BlockSpec((1, H), full)],
        out_specs=pl.BlockSpec((bm, H), row),
        out_shape=jax.ShapeDtypeStruct((n, H), jnp.float32),
    )(g1, g2, pe, ef, p['w2'], p['b2'].reshape(1, H), p['g'].reshape(1, H),
      p['be'].reshape(1, H))


def _node_update(d, agg, p, dual):
    n = d.shape[0]
    bm = _pick_bm(n)
    w1d = p['w1'][:H]
    w1a = p['w1'][H:]

    def body(d_ref, a_ref, w1d_ref, w1a_ref, b1_ref, w2_ref, b2_ref, g_ref,
             be_ref, o_ref):
        a = a_ref[0] + a_ref[1] if dual else a_ref[...]
        h = jax.nn.silu(jnp.dot(d_ref[...], w1d_ref[...],
                                preferred_element_type=jnp.float32)
                        + jnp.dot(a, w1a_ref[...],
                                  preferred_element_type=jnp.float32)
                        + b1_ref[...])
        h = jnp.dot(h, w2_ref[...], preferred_element_type=jnp.float32) \
            + b2_ref[...]
        o_ref[...] = d_ref[...] + _ln(h, g_ref[...], be_ref[...])

    full = lambda i: (0, 0)
    row = lambda i: (i, 0)
    if dual:
        a_spec = pl.BlockSpec((2, bm, H), lambda i: (0, i, 0))
    else:
        a_spec = pl.BlockSpec((bm, H), row)
    return pl.pallas_call(
        body,
        grid=(n // bm,),
        in_specs=[pl.BlockSpec((bm, H), row), a_spec,
                  pl.BlockSpec((H, H), full), pl.BlockSpec((H, H), full),
                  pl.BlockSpec((1, H), full), pl.BlockSpec((H, H), full),
                  pl.BlockSpec((1, H), full), pl.BlockSpec((1, H), full),
                  pl.BlockSpec((1, H), full)],
        out_specs=pl.BlockSpec((bm, H), row),
        out_shape=jax.ShapeDtypeStruct((n, H), jnp.float32),
    )(d, agg, w1d, w1a, p['b1'].reshape(1, H), p['w2'],
      p['b2'].reshape(1, H), p['g'].reshape(1, H), p['be'].reshape(1, H))


NB = 3


def _sc_gather2(ps, pd, src_idx, dst_idx, epad):
    per_w = epad // NW
    nch = per_w // CHUNK
    assert nch % NB == 0 and nch >= NB
    mesh = plsc.VectorSubcoreMesh(core_axis_name="c", subcore_axis_name="s")

    @functools.partial(
        pl.kernel, mesh=mesh,
        out_type=[jax.ShapeDtypeStruct((epad, H), jnp.float32)] * 2,
        scratch_types=[pltpu.VMEM((NB, CHUNK), jnp.int32),
                       pltpu.VMEM((NB, CHUNK), jnp.int32),
                       pltpu.VMEM((NB, CHUNK, H), jnp.float32),
                       pltpu.VMEM((NB, CHUNK, H), jnp.float32)]
                      + [pltpu.SemaphoreType.DMA] * (3 * NB))
    def k(ps_hbm, pd_hbm, si_hbm, di_hbm, o1_hbm, o2_hbm,
          i1, i2, r1, r2, *sems):
        isem = sems[0:NB]
        gsem = sems[NB:2 * NB]
        wsem = sems[2 * NB:3 * NB]
        wid = lax.axis_index("s") * NC + lax.axis_index("c")
        base0 = wid * per_w

        def fire_idx(j, b):
            base = base0 + j * CHUNK
            pltpu.async_copy(si_hbm.at[pl.ds(base, CHUNK)], i1.at[b],
                             isem[b])
            pltpu.async_copy(di_hbm.at[pl.ds(base, CHUNK)], i2.at[b],
                             isem[b])

        def fire_gather(b):
            pltpu.make_async_copy(si_hbm.at[pl.ds(0, CHUNK)], i1.at[b],
                                  isem[b]).wait()
            pltpu.make_async_copy(di_hbm.at[pl.ds(0, CHUNK)], i2.at[b],
                                  isem[b]).wait()
            pltpu.async_copy(ps_hbm.at[i1.at[b]], r1.at[b], gsem[b])
            pltpu.async_copy(pd_hbm.at[i2.at[b]], r2.at[b], gsem[b])

        def drain_wb(b):
            pltpu.make_async_copy(r1.at[b], o1_hbm.at[pl.ds(0, CHUNK)],
                                  wsem[b]).wait()
            pltpu.make_async_copy(r2.at[b], o2_hbm.at[pl.ds(0, CHUNK)],
                                  wsem[b]).wait()

        fire_idx(0, 0)
        fire_idx(1, 1)
        fire_gather(0)

        def group(jj, carry):
            for b in range(NB):
                j = jj * NB + b
                bn1 = (b + 1) % NB
                bn2 = (b + 2) % NB

                @pl.when(j + 2 < nch)
                def _():
                    fire_idx(j + 2, bn2)

                @pl.when(j + 1 < nch)
                def _():
                    @pl.when(j + 1 >= NB)
                    def _():
                        drain_wb(bn1)
                    fire_gather(bn1)

                pltpu.make_async_copy(ps_hbm.at[pl.ds(0, CHUNK)], r1.at[b],
                                      gsem[b]).wait()
                pltpu.make_async_copy(pd_hbm.at[pl.ds(0, CHUNK)], r2.at[b],
                                      gsem[b]).wait()
                base = base0 + j * CHUNK
                pltpu.async_copy(r1.at[b], o1_hbm.at[pl.ds(base, CHUNK)],
                                 wsem[b])
                pltpu.async_copy(r2.at[b], o2_hbm.at[pl.ds(base, CHUNK)],
                                 wsem[b])
            return carry

        lax.fori_loop(0, nch // NB, group, 0)
        for b in range(NB):
            drain_wb(b)

    return k(ps, pd, src_idx, dst_idx)


def _sc_segsum_dual(vals, dst_idx, nd, epad, zeros):
    ndb = (nd + 127) // 128 * 128 + 128
    CCH = 128
    NBL = 2
    per_w = epad // NW
    nch = per_w // CCH
    rpt_z = ndb // NS
    rpt_o = (nd // NS + 7) // 8 * 8
    rpt_last = nd - (NS - 1) * rpt_o
    assert rpt_last > 0 and rpt_o % 8 == 0 and rpt_last % 8 == 0
    mesh = plsc.VectorSubcoreMesh(core_axis_name="c", subcore_axis_name="s")

    assert nch >= 2

    @functools.partial(
        pl.kernel, mesh=mesh,
        out_type=jax.ShapeDtypeStruct((NC, nd, H), jnp.float32),
        scratch_types=[pltpu.VMEM((NBL, CCH), jnp.int32),
                       pltpu.VMEM((NBL, CCH, H), jnp.float32),
                       pltpu.VMEM_SHARED((ndb, H), jnp.float32)]
                      + [pltpu.SemaphoreType.DMA] * NBL)
    def k(v_hbm, di_hbm, z_hbm, out_hbm, idx, buf, shared, *lsem):
        c = lax.axis_index("c")
        s = lax.axis_index("s")
        wid = s * NC + c
        base0 = wid * per_w

        def fire_load(j, b):
            base = base0 + j * CCH
            pltpu.async_copy(di_hbm.at[pl.ds(base, CCH)], idx.at[b],
                             lsem[b])
            pltpu.async_copy(v_hbm.at[pl.ds(base, CCH)], buf.at[b],
                             lsem[b])

        def consume(b):
            pltpu.make_async_copy(di_hbm.at[pl.ds(0, CCH)], idx.at[b],
                                  lsem[b]).wait()
            pltpu.make_async_copy(v_hbm.at[pl.ds(0, CCH)], buf.at[b],
                                  lsem[b]).wait()
            pltpu.sync_copy(buf.at[b], shared.at[idx.at[b]], add=True)

        pltpu.sync_copy(z_hbm.at[pl.ds(s * rpt_z, rpt_z)],
                        shared.at[pl.ds(s * rpt_z, rpt_z)])
        plsc.subcore_barrier()
        fire_load(0, 0)

        def group(jj, carry):
            for b in range(NBL):
                j = jj * NBL + b

                @pl.when(j + 1 < nch)
                def _():
                    fire_load(j + 1, (b + 1) % NBL)

                consume(b)
            return carry

        lax.fori_loop(0, nch // NBL, group, 0)
        if nch % NBL:
            consume((nch - 1) % NBL)
        plsc.subcore_barrier()

        @pl.when(s < NS - 1)
        def _():
            pltpu.sync_copy(shared.at[pl.ds(s * rpt_o, rpt_o)],
                            out_hbm.at[c, pl.ds(s * rpt_o, rpt_o)])

        @pl.when(s == NS - 1)
        def _():
            pltpu.sync_copy(shared.at[pl.ds((NS - 1) * rpt_o, rpt_last)],
                            out_hbm.at[c, pl.ds((NS - 1) * rpt_o, rpt_last)])

    return k(vals, dst_idx, zeros)


def _sc_segsum_chunked(vals, dst_idx, epad, zeros, csize, chunks_per_sc):
    cbuf = csize + 128
    CCH = 96
    NBL = 2
    per_t = epad // NS
    nch = per_t // CCH
    rpt_z = cbuf // NS
    rpt_o = csize // NS
    assert rpt_z % 8 == 0 and rpt_o % 8 == 0
    assert per_t % CCH == 0 and nch >= 2
    nout = NC * chunks_per_sc * csize
    mesh = plsc.VectorSubcoreMesh(core_axis_name="c", subcore_axis_name="s")

    @functools.partial(
        pl.kernel, mesh=mesh,
        out_type=jax.ShapeDtypeStruct((nout, H), jnp.float32),
        scratch_types=[pltpu.VMEM((NBL, CCH), jnp.int32),
                       pltpu.VMEM((NBL, CCH), jnp.int32),
                       pltpu.VMEM((NBL, CCH, H), jnp.float32),
                       pltpu.VMEM_SHARED((cbuf, H), jnp.float32)]
                      + [pltpu.SemaphoreType.DMA] * NBL)
    def k(v_hbm, di_hbm, z_hbm, out_hbm, idx, idx2, buf, shared, *lsem):
        c = lax.axis_index("c")
        s = lax.axis_index("s")
        base0 = s * per_t

        def fire_load(j, b):
            base = base0 + j * CCH
            pltpu.async_copy(di_hbm.at[pl.ds(base, CCH)], idx.at[b],
                             lsem[b])
            pltpu.async_copy(v_hbm.at[pl.ds(base, CCH)], buf.at[b],
                             lsem[b])

        def chunk_loop(ci, carry):
            cbase = (c * chunks_per_sc + ci) * csize

            def consume(b):
                pltpu.make_async_copy(di_hbm.at[pl.ds(0, CCH)], idx.at[b],
                                      lsem[b]).wait()
                pltpu.make_async_copy(v_hbm.at[pl.ds(0, CCH)], buf.at[b],
                                      lsem[b]).wait()
                for r in range(CCH // 16):
                    v = idx[b, pl.ds(r * 16, 16)] - cbase
                    ok = (v >= 0) & (v < csize)
                    idx2[b, pl.ds(r * 16, 16)] = jnp.where(ok, v, csize)
                pltpu.sync_copy(buf.at[b], shared.at[idx2.at[b]],
                                add=True)

            pltpu.sync_copy(z_hbm.at[pl.ds(s * rpt_z, rpt_z)],
                            shared.at[pl.ds(s * rpt_z, rpt_z)])
            fire_load(0, 0)
            plsc.subcore_barrier()

            def group(jj, carry2):
                for b in range(NBL):
                    j = jj * NBL + b

                    @pl.when(j + 1 < nch)
                    def _():
                        fire_load(j + 1, (b + 1) % NBL)

                    consume(b)
                return carry2

            lax.fori_loop(0, nch // NBL, group, 0)
            if nch % NBL:
                consume((nch - 1) % NBL)
            plsc.subcore_barrier()
            pltpu.sync_copy(shared.at[pl.ds(s * rpt_o, rpt_o)],
                            out_hbm.at[pl.ds(cbase + s * rpt_o, rpt_o)])
            plsc.subcore_barrier()
            return carry

        lax.fori_loop(0, chunks_per_sc, chunk_loop, 0)

    return k(vals, dst_idx, zeros)


def _pad_edges(e):
    step = NW * CHUNK * NB
    return (e + step - 1) // step * step


def _interaction_mesh_dst(src_feat, dst_feat, src_idx, dst_idx, dst_trash,
                          efeat, p, nd, epad, zeros, fused_psd=False):
    w1 = p['edge']['w1']
    if fused_psd:
        psd = _matmul_bias(src_feat, jnp.concatenate([w1[:H], w1[H:2 * H]],
                                                     axis=1),
                           jnp.zeros((2 * H,), jnp.float32))
        ps, pd = psd[:, :H], psd[:, H:]
    else:
        zb = jnp.zeros((H,), jnp.float32)
        ps = _matmul_bias(src_feat, w1[:H], zb)
        pd = _matmul_bias(dst_feat, w1[H:2 * H], zb)
    pe = _matmul_bias(efeat, w1[2 * H:], p['edge']['b1'])
    g1, g2 = _sc_gather2(ps, pd, src_idx, dst_idx, epad)
    e_new = _edge_update(g1, g2, pe, efeat, p['edge'])
    agg = _sc_segsum_dual(e_new, dst_trash, nd, epad, zeros)
    d_new = _node_update(dst_feat, agg, p['node'], dual=True)
    return d_new, e_new


def kernel(x, mesh_x, g2m_x_s, g2m_edge_index, g2m_edge_attr,
           mesh_edge_index, mesh_edge_attr, m2g_edge_index, m2g_edge_attr,
           params):
    n_grid = x.shape[1]
    n_mesh = mesh_x.shape[0]

    def prep(ei, ea, n_dst_trash):
        e = ei.shape[1]
        ep = _pad_edges(e)
        src = jnp.concatenate([ei[0], jnp.zeros((ep - e,), jnp.int32)])
        dst = jnp.concatenate([ei[1], jnp.zeros((ep - e,), jnp.int32)])
        dtr = jnp.concatenate(
            [ei[1], jnp.full((ep - e,), n_dst_trash, jnp.int32)])
        eap = jnp.concatenate(
            [ea, jnp.zeros((ep - e, ea.shape[1]), jnp.float32)])
        return src, dst, dtr, eap, ep

    g2m_src, g2m_dst, g2m_dtr, g2m_ea, ep_g2m = prep(
        g2m_edge_index, g2m_edge_attr, n_mesh)
    mm_src, mm_dst, mm_dtr, mm_ea, ep_mm = prep(
        mesh_edge_index, mesh_edge_attr, n_mesh)
    m2g_src, m2g_dst, m2g_dtr, m2g_ea, ep_m2g = prep(
        m2g_edge_index, m2g_edge_attr, 1 << 29)

    zeros_mesh = jnp.zeros((n_mesh + 16, H), jnp.float32)
    csize = 12544
    zeros_grid = jnp.zeros((csize + 128, H), jnp.float32)

    grid_in = jnp.concatenate([jnp.squeeze(x, 0), g2m_x_s], axis=-1)
    grid = _mlp(grid_in, params['grid_embed'])
    pme = params['mesh_embed']
    mesh = _mlp(mesh_x, {'w1': pme['w1'][-mesh_x.shape[1]:], 'b1': pme['b1'],
                         'w2': pme['w2'], 'b2': pme['b2'], 'g': pme['g'],
                         'be': pme['be']})
    e = _mlp(g2m_ea, params['g2m_edge_embed'])
    mesh, _ = _interaction_mesh_dst(
        grid, mesh, g2m_src, g2m_dst, g2m_dtr, e, params['g2m_gnn'],
        n_mesh, ep_g2m, zeros_mesh)
    grid = _mlp(grid, params['grid_model'], residual=True)

    em = _mlp(mm_ea, params['proc_edge_embed'])
    for p in params['proc']:
        mesh, em = _interaction_mesh_dst(
            mesh, mesh, mm_src, mm_dst, mm_dtr, em, p,
            n_mesh, ep_mm, zeros_mesh, fused_psd=True)

    eg = _mlp(m2g_ea, params['m2g_edge_embed'])
    pg = params['m2g_gnn']
    w1 = pg['edge']['w1']
    zb = jnp.zeros((H,), jnp.float32)
    ps = _matmul_bias(mesh, w1[:H], zb)
    pd = _matmul_bias(grid, w1[H:2 * H], zb)
    pe = _matmul_bias(eg, w1[2 * H:], pg['edge']['b1'])
    g1, g2 = _sc_gather2(ps, pd, m2g_src, m2g_dst, ep_m2g)
    e_new = _edge_update(g1, g2, pe, eg, pg['edge'])
    agg = _sc_segsum_chunked(e_new, m2g_dtr, ep_m2g, zeros_grid,
                             csize, chunks_per_sc=2)
    grid = _node_update(grid, agg[:n_grid], pg['node'], dual=False)

    out = _mlp(grid, params['final'], norm=False)
    return out[None]


# speedup vs baseline: 1.1065x; 1.1065x over previous
import functools

import jax
import jax.numpy as jnp
from jax import lax
from jax.experimental import pallas as pl
from jax.experimental.pallas import tpu as pltpu
from jax.experimental.pallas import tpu_sc as plsc

H = 128
NC = 2
NS = 16
NW = NC * NS
CHUNK = 128


def _pick_bm(n, cap=2048):
    for bm in (2048, 1024, 1000, 512, 500, 256, 250, 200, 128, 125, 100, 64,
               50, 40, 32, 25, 16, 10, 8, 5, 4, 2, 1):
        if bm <= cap and n % bm == 0:
            return bm
    return 1


def _ln(h, g, be):
    mu = jnp.mean(h, axis=-1, keepdims=True)
    var = jnp.mean((h - mu) ** 2, axis=-1, keepdims=True)
    return (h - mu) * lax.rsqrt(var + 1e-5) * g + be


def _matmul_bias(x, w, b):
    n, k = x.shape
    m = w.shape[1]
    bm = _pick_bm(n)

    def body(x_ref, w_ref, b_ref, o_ref):
        o_ref[...] = jnp.dot(x_ref[...], w_ref[...],
                             preferred_element_type=jnp.float32) + b_ref[...]

    return pl.pallas_call(
        body,
        grid=(n // bm,),
        in_specs=[pl.BlockSpec((bm, k), lambda i: (i, 0)),
                  pl.BlockSpec((k, m), lambda i: (0, 0)),
                  pl.BlockSpec((1, m), lambda i: (0, 0))],
        out_specs=pl.BlockSpec((bm, m), lambda i: (i, 0)),
        out_shape=jax.ShapeDtypeStruct((n, m), jnp.float32),
    )(x, w, b.reshape(1, m))


def _mlp(x, p, residual=False, norm=True):
    n, k = x.shape
    m = p['w2'].shape[1]
    bm = _pick_bm(n)
    g = p['g'].reshape(1, m) if norm else jnp.zeros((1, m), jnp.float32)
    be = p['be'].reshape(1, m) if norm else jnp.zeros((1, m), jnp.float32)

    def body(x_ref, w1_ref, b1_ref, w2_ref, b2_ref, g_ref, be_ref, o_ref):
        h = jax.nn.silu(jnp.dot(x_ref[...], w1_ref[...],
                                preferred_element_type=jnp.float32)
                        + b1_ref[...])
        h = jnp.dot(h, w2_ref[...], preferred_element_type=jnp.float32) \
            + b2_ref[...]
        if norm:
            h = _ln(h, g_ref[...], be_ref[...])
        if residual:
            h = h + x_ref[...]
        o_ref[...] = h

    hdim = p['w1'].shape[1]
    return pl.pallas_call(
        body,
        grid=(n // bm,),
        in_specs=[pl.BlockSpec((bm, k), lambda i: (i, 0)),
                  pl.BlockSpec((k, hdim), lambda i: (0, 0)),
                  pl.BlockSpec((1, hdim), lambda i: (0, 0)),
                  pl.BlockSpec((hdim, m), lambda i: (0, 0)),
                  pl.BlockSpec((1, m), lambda i: (0, 0)),
                  pl.BlockSpec((1, m), lambda i: (0, 0)),
                  pl.BlockSpec((1, m), lambda i: (0, 0))],
        out_specs=pl.BlockSpec((bm, m), lambda i: (i, 0)),
        out_shape=jax.ShapeDtypeStruct((n, m), jnp.float32),
    )(x, p['w1'], p['b1'].reshape(1, hdim), p['w2'], p['b2'].reshape(1, m),
      g, be)


def _edge_update(g, ef, p):
    n = g.shape[0]
    bm = _pick_bm(n)
    w1e = p['w1'][2 * H:]

    def body(g_ref, ef_ref, w1e_ref, b1_ref, w2_ref, b2_ref, gn_ref, be_ref,
             o_ref):
        h = jax.nn.silu(g_ref[...]
                        + jnp.dot(ef_ref[...], w1e_ref[...],
                                  preferred_element_type=jnp.float32)
                        + b1_ref[...])
        h = jnp.dot(h, w2_ref[...], preferred_element_type=jnp.float32) \
            + b2_ref[...]
        o_ref[...] = ef_ref[...] + _ln(h, gn_ref[...], be_ref[...])

    full = lambda i: (0, 0)
    row = lambda i: (i, 0)
    return pl.pallas_call(
        body,
        grid=(n // bm,),
        in_specs=[pl.BlockSpec((bm, H), row), pl.BlockSpec((bm, H), row),
                  pl.BlockSpec((H, H), full),
                  pl.BlockSpec((1, H), full),
                  pl.BlockSpec((H, H), full),
                  pl.BlockSpec((1, H), full),
                  pl.BlockSpec((1, H), full),
                  pl.BlockSpec((1, H), full)],
        out_specs=pl.BlockSpec((bm, H), row),
        out_shape=jax.ShapeDtypeStruct((n, H), jnp.float32),
    )(g, ef, w1e, p['b1'].reshape(1, H), p['w2'], p['b2'].reshape(1, H),
      p['g'].reshape(1, H), p['be'].reshape(1, H))


def _node_update(d, agg, p, dual):
    n = d.shape[0]
    bm = _pick_bm(n)
    w1d = p['w1'][:H]
    w1a = p['w1'][H:]

    def body(d_ref, a_ref, w1d_ref, w1a_ref, b1_ref, w2_ref, b2_ref, g_ref,
             be_ref, o_ref):
        a = a_ref[0] + a_ref[1] if dual else a_ref[...]
        h = jax.nn.silu(jnp.dot(d_ref[...], w1d_ref[...],
                                preferred_element_type=jnp.float32)
                        + jnp.dot(a, w1a_ref[...],
                                  preferred_element_type=jnp.float32)
                        + b1_ref[...])
        h = jnp.dot(h, w2_ref[...], preferred_element_type=jnp.float32) \
            + b2_ref[...]
        o_ref[...] = d_ref[...] + _ln(h, g_ref[...], be_ref[...])

    full = lambda i: (0, 0)
    row = lambda i: (i, 0)
    if dual:
        a_spec = pl.BlockSpec((2, bm, H), lambda i: (0, i, 0))
    else:
        a_spec = pl.BlockSpec((bm, H), row)
    return pl.pallas_call(
        body,
        grid=(n // bm,),
        in_specs=[pl.BlockSpec((bm, H), row), a_spec,
                  pl.BlockSpec((H, H), full), pl.BlockSpec((H, H), full),
                  pl.BlockSpec((1, H), full), pl.BlockSpec((H, H), full),
                  pl.BlockSpec((1, H), full), pl.BlockSpec((1, H), full),
                  pl.BlockSpec((1, H), full)],
        out_specs=pl.BlockSpec((bm, H), row),
        out_shape=jax.ShapeDtypeStruct((n, H), jnp.float32),
    )(d, agg, w1d, w1a, p['b1'].reshape(1, H), p['w2'],
      p['b2'].reshape(1, H), p['g'].reshape(1, H), p['be'].reshape(1, H))


NB = 3


def _sc_gather_add(ps, pd, src_idx, dst_idx, epad):
    per_w = epad // NW
    nch = per_w // CHUNK
    assert nch % NB == 0 and nch >= NB
    mesh = plsc.VectorSubcoreMesh(core_axis_name="c", subcore_axis_name="s")

    @functools.partial(
        pl.kernel, mesh=mesh,
        out_type=jax.ShapeDtypeStruct((epad, H), jnp.float32),
        scratch_types=[pltpu.VMEM((NB, CHUNK), jnp.int32),
                       pltpu.VMEM((NB, CHUNK), jnp.int32),
                       pltpu.VMEM((NB, CHUNK, H), jnp.float32)]
                      + [pltpu.SemaphoreType.DMA] * (4 * NB))
    def k(ps_hbm, pd_hbm, si_hbm, di_hbm, o_hbm,
          i1, i2, r1, *sems):
        isem = sems[0:NB]
        g1sem = sems[NB:2 * NB]
        g2sem = sems[2 * NB:3 * NB]
        wsem = sems[3 * NB:4 * NB]
        wid = lax.axis_index("s") * NC + lax.axis_index("c")
        base0 = wid * per_w

        def fire_idx(j, b):
            base = base0 + j * CHUNK
            pltpu.async_copy(si_hbm.at[pl.ds(base, CHUNK)], i1.at[b],
                             isem[b])
            pltpu.async_copy(di_hbm.at[pl.ds(base, CHUNK)], i2.at[b],
                             isem[b])

        def fire_g1(b):
            pltpu.make_async_copy(si_hbm.at[pl.ds(0, CHUNK)], i1.at[b],
                                  isem[b]).wait()
            pltpu.make_async_copy(di_hbm.at[pl.ds(0, CHUNK)], i2.at[b],
                                  isem[b]).wait()
            pltpu.async_copy(ps_hbm.at[i1.at[b]], r1.at[b], g1sem[b])

        def fire_g2(b):
            pltpu.make_async_copy(ps_hbm.at[pl.ds(0, CHUNK)], r1.at[b],
                                  g1sem[b]).wait()
            pltpu.async_copy(pd_hbm.at[i2.at[b]], r1.at[b], g2sem[b],
                             add=True)

        def drain_wb(b):
            pltpu.make_async_copy(r1.at[b], o_hbm.at[pl.ds(0, CHUNK)],
                                  wsem[b]).wait()

        fire_idx(0, 0)
        fire_idx(1, 1)
        fire_idx(2, 2)
        fire_g1(0)
        fire_g2(0)
        fire_g1(1)

        def group(jj, carry):
            for b in range(NB):
                j = jj * NB + b
                bn1 = (b + 1) % NB
                bn2 = (b + 2) % NB

                pltpu.make_async_copy(pd_hbm.at[pl.ds(0, CHUNK)], r1.at[b],
                                      g2sem[b]).wait()
                base = base0 + j * CHUNK
                pltpu.async_copy(r1.at[b], o_hbm.at[pl.ds(base, CHUNK)],
                                 wsem[b])

                @pl.when(j + 3 < nch)
                def _():
                    fire_idx(j + 3, b)

                @pl.when(j + 2 < nch)
                def _():
                    @pl.when(j + 2 >= NB)
                    def _():
                        drain_wb(bn2)
                    fire_g1(bn2)

                @pl.when(j + 1 < nch)
                def _():
                    fire_g2(bn1)
            return carry

        lax.fori_loop(0, nch // NB, group, 0)
        for b in range(NB):
            drain_wb(b)

    return k(ps, pd, src_idx, dst_idx)


def _sc_segsum_dual(vals, dst_idx, nd, epad, zeros):
    ndb = (nd + 127) // 128 * 128 + 128
    CCH = 128
    NBL = 2
    per_w = epad // NW
    nch = per_w // CCH
    rpt_z = ndb // NS
    rpt_o = (nd // NS + 7) // 8 * 8
    rpt_last = nd - (NS - 1) * rpt_o
    assert rpt_last > 0 and rpt_o % 8 == 0 and rpt_last % 8 == 0
    mesh = plsc.VectorSubcoreMesh(core_axis_name="c", subcore_axis_name="s")

    assert nch >= 2

    @functools.partial(
        pl.kernel, mesh=mesh,
        out_type=jax.ShapeDtypeStruct((NC, nd, H), jnp.float32),
        scratch_types=[pltpu.VMEM((NBL, CCH), jnp.int32),
                       pltpu.VMEM((NBL, CCH, H), jnp.float32),
                       pltpu.VMEM_SHARED((ndb, H), jnp.float32)]
                      + [pltpu.SemaphoreType.DMA] * NBL)
    def k(v_hbm, di_hbm, z_hbm, out_hbm, idx, buf, shared, *lsem):
        c = lax.axis_index("c")
        s = lax.axis_index("s")
        wid = s * NC + c
        base0 = wid * per_w

        def fire_load(j, b):
            base = base0 + j * CCH
            pltpu.async_copy(di_hbm.at[pl.ds(base, CCH)], idx.at[b],
                             lsem[b])
            pltpu.async_copy(v_hbm.at[pl.ds(base, CCH)], buf.at[b],
                             lsem[b])

        def consume(b):
            pltpu.make_async_copy(di_hbm.at[pl.ds(0, CCH)], idx.at[b],
                                  lsem[b]).wait()
            pltpu.make_async_copy(v_hbm.at[pl.ds(0, CCH)], buf.at[b],
                                  lsem[b]).wait()
            pltpu.sync_copy(buf.at[b], shared.at[idx.at[b]], add=True)

        pltpu.sync_copy(z_hbm.at[pl.ds(s * rpt_z, rpt_z)],
                        shared.at[pl.ds(s * rpt_z, rpt_z)])
        plsc.subcore_barrier()
        fire_load(0, 0)

        def group(jj, carry):
            for b in range(NBL):
                j = jj * NBL + b

                @pl.when(j + 1 < nch)
                def _():
                    fire_load(j + 1, (b + 1) % NBL)

                consume(b)
            return carry

        lax.fori_loop(0, nch // NBL, group, 0)
        if nch % NBL:
            consume((nch - 1) % NBL)
        plsc.subcore_barrier()

        @pl.when(s < NS - 1)
        def _():
            pltpu.sync_copy(shared.at[pl.ds(s * rpt_o, rpt_o)],
                            out_hbm.at[c, pl.ds(s * rpt_o, rpt_o)])

        @pl.when(s == NS - 1)
        def _():
            pltpu.sync_copy(shared.at[pl.ds((NS - 1) * rpt_o, rpt_last)],
                            out_hbm.at[c, pl.ds((NS - 1) * rpt_o, rpt_last)])

    return k(vals, dst_idx, zeros)


def _sc_segsum_chunked(vals, dst_idx, epad, zeros, csize, chunks_per_sc):
    cbuf = csize + 128
    CCH = 96
    NBL = 2
    per_t = epad // NS
    nch = per_t // CCH
    rpt_z = cbuf // NS
    rpt_o = csize // NS
    assert rpt_z % 8 == 0 and rpt_o % 8 == 0
    assert per_t % CCH == 0 and nch >= 2
    nout = NC * chunks_per_sc * csize
    mesh = plsc.VectorSubcoreMesh(core_axis_name="c", subcore_axis_name="s")

    @functools.partial(
        pl.kernel, mesh=mesh,
        out_type=jax.ShapeDtypeStruct((nout, H), jnp.float32),
        scratch_types=[pltpu.VMEM((NBL, CCH), jnp.int32),
                       pltpu.VMEM((NBL, CCH), jnp.int32),
                       pltpu.VMEM((NBL, CCH, H), jnp.float32),
                       pltpu.VMEM_SHARED((cbuf, H), jnp.float32)]
                      + [pltpu.SemaphoreType.DMA] * NBL)
    def k(v_hbm, di_hbm, z_hbm, out_hbm, idx, idx2, buf, shared, *lsem):
        c = lax.axis_index("c")
        s = lax.axis_index("s")
        base0 = s * per_t

        def fire_load(j, b):
            base = base0 + j * CCH
            pltpu.async_copy(di_hbm.at[pl.ds(base, CCH)], idx.at[b],
                             lsem[b])
            pltpu.async_copy(v_hbm.at[pl.ds(base, CCH)], buf.at[b],
                             lsem[b])

        def chunk_loop(ci, carry):
            cbase = (c * chunks_per_sc + ci) * csize

            def consume(b):
                pltpu.make_async_copy(di_hbm.at[pl.ds(0, CCH)], idx.at[b],
                                      lsem[b]).wait()
                pltpu.make_async_copy(v_hbm.at[pl.ds(0, CCH)], buf.at[b],
                                      lsem[b]).wait()
                for r in range(CCH // 16):
                    v = idx[b, pl.ds(r * 16, 16)] - cbase
                    ok = (v >= 0) & (v < csize)
                    idx2[b, pl.ds(r * 16, 16)] = jnp.where(ok, v, csize)
                pltpu.sync_copy(buf.at[b], shared.at[idx2.at[b]],
                                add=True)

            pltpu.sync_copy(z_hbm.at[pl.ds(s * rpt_z, rpt_z)],
                            shared.at[pl.ds(s * rpt_z, rpt_z)])
            fire_load(0, 0)
            plsc.subcore_barrier()

            def group(jj, carry2):
                for b in range(NBL):
                    j = jj * NBL + b

                    @pl.when(j + 1 < nch)
                    def _():
                        fire_load(j + 1, (b + 1) % NBL)

                    consume(b)
                return carry2

            lax.fori_loop(0, nch // NBL, group, 0)
            if nch % NBL:
                consume((nch - 1) % NBL)
            plsc.subcore_barrier()
            pltpu.sync_copy(shared.at[pl.ds(s * rpt_o, rpt_o)],
                            out_hbm.at[pl.ds(cbase + s * rpt_o, rpt_o)])
            plsc.subcore_barrier()
            return carry

        lax.fori_loop(0, chunks_per_sc, chunk_loop, 0)

    return k(vals, dst_idx, zeros)


def _pad_edges(e):
    step = NW * CHUNK * NB
    return (e + step - 1) // step * step


def _interaction_mesh_dst(src_feat, dst_feat, src_idx, dst_idx, dst_trash,
                          efeat, p, nd, epad, zeros, fused_psd=False):
    w1 = p['edge']['w1']
    if fused_psd:
        psd = _matmul_bias(src_feat, jnp.concatenate([w1[:H], w1[H:2 * H]],
                                                     axis=1),
                           jnp.zeros((2 * H,), jnp.float32))
        ps, pd = psd[:, :H], psd[:, H:]
    else:
        zb = jnp.zeros((H,), jnp.float32)
        ps = _matmul_bias(src_feat, w1[:H], zb)
        pd = _matmul_bias(dst_feat, w1[H:2 * H], zb)
    g = _sc_gather_add(ps, pd, src_idx, dst_idx, epad)
    e_new = _edge_update(g, efeat, p['edge'])
    agg = _sc_segsum_dual(e_new, dst_trash, nd, epad, zeros)
    d_new = _node_update(dst_feat, agg, p['node'], dual=True)
    return d_new, e_new


def kernel(x, mesh_x, g2m_x_s, g2m_edge_index, g2m_edge_attr,
           mesh_edge_index, mesh_edge_attr, m2g_edge_index, m2g_edge_attr,
           params):
    n_grid = x.shape[1]
    n_mesh = mesh_x.shape[0]

    def prep(ei, ea, n_dst_trash):
        e = ei.shape[1]
        ep = _pad_edges(e)
        src = jnp.concatenate([ei[0], jnp.zeros((ep - e,), jnp.int32)])
        dst = jnp.concatenate([ei[1], jnp.zeros((ep - e,), jnp.int32)])
        dtr = jnp.concatenate(
            [ei[1], jnp.full((ep - e,), n_dst_trash, jnp.int32)])
        eap = jnp.concatenate(
            [ea, jnp.zeros((ep - e, ea.shape[1]), jnp.float32)])
        return src, dst, dtr, eap, ep

    g2m_src, g2m_dst, g2m_dtr, g2m_ea, ep_g2m = prep(
        g2m_edge_index, g2m_edge_attr, n_mesh)
    mm_src, mm_dst, mm_dtr, mm_ea, ep_mm = prep(
        mesh_edge_index, mesh_edge_attr, n_mesh)
    m2g_src, m2g_dst, m2g_dtr, m2g_ea, ep_m2g = prep(
        m2g_edge_index, m2g_edge_attr, 1 << 29)

    zeros_mesh = jnp.zeros((n_mesh + 16, H), jnp.float32)
    csize = 12544
    zeros_grid = jnp.zeros((csize + 128, H), jnp.float32)

    grid_in = jnp.concatenate([jnp.squeeze(x, 0), g2m_x_s], axis=-1)
    grid = _mlp(grid_in, params['grid_embed'])
    pme = params['mesh_embed']
    mesh = _mlp(mesh_x, {'w1': pme['w1'][-mesh_x.shape[1]:], 'b1': pme['b1'],
                         'w2': pme['w2'], 'b2': pme['b2'], 'g': pme['g'],
                         'be': pme['be']})
    e = _mlp(g2m_ea, params['g2m_edge_embed'])
    mesh, _ = _interaction_mesh_dst(
        grid, mesh, g2m_src, g2m_dst, g2m_dtr, e, params['g2m_gnn'],
        n_mesh, ep_g2m, zeros_mesh)
    grid = _mlp(grid, params['grid_model'], residual=True)

    em = _mlp(mm_ea, params['proc_edge_embed'])
    for p in params['proc']:
        mesh, em = _interaction_mesh_dst(
            mesh, mesh, mm_src, mm_dst, mm_dtr, em, p,
            n_mesh, ep_mm, zeros_mesh, fused_psd=True)

    eg = _mlp(m2g_ea, params['m2g_edge_embed'])
    pg = params['m2g_gnn']
    w1 = pg['edge']['w1']
    zb = jnp.zeros((H,), jnp.float32)
    ps = _matmul_bias(mesh, w1[:H], zb)
    pd = _matmul_bias(grid, w1[H:2 * H], zb)
    g = _sc_gather_add(ps, pd, m2g_src, m2g_dst, ep_m2g)
    e_new = _edge_update(g, eg, pg['edge'])
    agg = _sc_segsum_chunked(e_new, m2g_dtr, ep_m2g, zeros_grid,
                             csize, chunks_per_sc=2)
    grid = _node_update(grid, agg[:n_grid], pg['node'], dual=False)

    out = _mlp(grid, params['final'], norm=False)
    return out[None]


# speedup vs baseline: 1.1381x; 1.0286x over previous
import functools

import jax
import jax.numpy as jnp
from jax import lax
from jax.experimental import pallas as pl
from jax.experimental.pallas import tpu as pltpu
from jax.experimental.pallas import tpu_sc as plsc

H = 128
NC = 2
NS = 16
NW = NC * NS
CHUNK = 128


def _pick_bm(n, cap=2048):
    for bm in (2048, 1024, 1000, 512, 500, 256, 250, 200, 128, 125, 100, 64,
               50, 40, 32, 25, 16, 10, 8, 5, 4, 2, 1):
        if bm <= cap and n % bm == 0:
            return bm
    return 1


def _ln(h, g, be):
    mu = jnp.mean(h, axis=-1, keepdims=True)
    var = jnp.mean((h - mu) ** 2, axis=-1, keepdims=True)
    return (h - mu) * lax.rsqrt(var + 1e-5) * g + be


def _matmul_bias(x, w, b):
    n, k = x.shape
    m = w.shape[1]
    bm = _pick_bm(n)

    def body(x_ref, w_ref, b_ref, o_ref):
        o_ref[...] = jnp.dot(x_ref[...], w_ref[...],
                             preferred_element_type=jnp.float32) + b_ref[...]

    return pl.pallas_call(
        body,
        grid=(n // bm,),
        in_specs=[pl.BlockSpec((bm, k), lambda i: (i, 0)),
                  pl.BlockSpec((k, m), lambda i: (0, 0)),
                  pl.BlockSpec((1, m), lambda i: (0, 0))],
        out_specs=pl.BlockSpec((bm, m), lambda i: (i, 0)),
        out_shape=jax.ShapeDtypeStruct((n, m), jnp.float32),
    )(x, w, b.reshape(1, m))


def _mlp(x, p, residual=False, norm=True):
    n, k = x.shape
    m = p['w2'].shape[1]
    bm = _pick_bm(n)
    g = p['g'].reshape(1, m) if norm else jnp.zeros((1, m), jnp.float32)
    be = p['be'].reshape(1, m) if norm else jnp.zeros((1, m), jnp.float32)

    def body(x_ref, w1_ref, b1_ref, w2_ref, b2_ref, g_ref, be_ref, o_ref):
        h = jax.nn.silu(jnp.dot(x_ref[...], w1_ref[...],
                                preferred_element_type=jnp.float32)
                        + b1_ref[...])
        h = jnp.dot(h, w2_ref[...], preferred_element_type=jnp.float32) \
            + b2_ref[...]
        if norm:
            h = _ln(h, g_ref[...], be_ref[...])
        if residual:
            h = h + x_ref[...]
        o_ref[...] = h

    hdim = p['w1'].shape[1]
    return pl.pallas_call(
        body,
        grid=(n // bm,),
        in_specs=[pl.BlockSpec((bm, k), lambda i: (i, 0)),
                  pl.BlockSpec((k, hdim), lambda i: (0, 0)),
                  pl.BlockSpec((1, hdim), lambda i: (0, 0)),
                  pl.BlockSpec((hdim, m), lambda i: (0, 0)),
                  pl.BlockSpec((1, m), lambda i: (0, 0)),
                  pl.BlockSpec((1, m), lambda i: (0, 0)),
                  pl.BlockSpec((1, m), lambda i: (0, 0))],
        out_specs=pl.BlockSpec((bm, m), lambda i: (i, 0)),
        out_shape=jax.ShapeDtypeStruct((n, m), jnp.float32),
    )(x, p['w1'], p['b1'].reshape(1, hdim), p['w2'], p['b2'].reshape(1, m),
      g, be)


def _edge_update(g, ef, p):
    n = g.shape[0]
    bm = _pick_bm(n)
    w1e = p['w1'][2 * H:]

    def body(g_ref, ef_ref, w1e_ref, b1_ref, w2_ref, b2_ref, gn_ref, be_ref,
             o_ref):
        h = jax.nn.silu(g_ref[...]
                        + jnp.dot(ef_ref[...], w1e_ref[...],
                                  preferred_element_type=jnp.float32)
                        + b1_ref[...])
        h = jnp.dot(h, w2_ref[...], preferred_element_type=jnp.float32) \
            + b2_ref[...]
        o_ref[...] = ef_ref[...] + _ln(h, gn_ref[...], be_ref[...])

    full = lambda i: (0, 0)
    row = lambda i: (i, 0)
    return pl.pallas_call(
        body,
        grid=(n // bm,),
        in_specs=[pl.BlockSpec((bm, H), row), pl.BlockSpec((bm, H), row),
                  pl.BlockSpec((H, H), full),
                  pl.BlockSpec((1, H), full),
                  pl.BlockSpec((H, H), full),
                  pl.BlockSpec((1, H), full),
                  pl.BlockSpec((1, H), full),
                  pl.BlockSpec((1, H), full)],
        out_specs=pl.BlockSpec((bm, H), row),
        out_shape=jax.ShapeDtypeStruct((n, H), jnp.float32),
    )(g, ef, w1e, p['b1'].reshape(1, H), p['w2'], p['b2'].reshape(1, H),
      p['g'].reshape(1, H), p['be'].reshape(1, H))


def _node_update(d, agg, p, dual):
    n = d.shape[0]
    bm = _pick_bm(n)
    w1d = p['w1'][:H]
    w1a = p['w1'][H:]

    def body(d_ref, a_ref, w1d_ref, w1a_ref, b1_ref, w2_ref, b2_ref, g_ref,
             be_ref, o_ref):
        a = a_ref[0] + a_ref[1] if dual else a_ref[...]
        h = jax.nn.silu(jnp.dot(d_ref[...], w1d_ref[...],
                                preferred_element_type=jnp.float32)
                        + jnp.dot(a, w1a_ref[...],
                                  preferred_element_type=jnp.float32)
                        + b1_ref[...])
        h = jnp.dot(h, w2_ref[...], preferred_element_type=jnp.float32) \
            + b2_ref[...]
        o_ref[...] = d_ref[...] + _ln(h, g_ref[...], be_ref[...])

    full = lambda i: (0, 0)
    row = lambda i: (i, 0)
    if dual:
        a_spec = pl.BlockSpec((2, bm, H), lambda i: (0, i, 0))
    else:
        a_spec = pl.BlockSpec((bm, H), row)
    return pl.pallas_call(
        body,
        grid=(n // bm,),
        in_specs=[pl.BlockSpec((bm, H), row), a_spec,
                  pl.BlockSpec((H, H), full), pl.BlockSpec((H, H), full),
                  pl.BlockSpec((1, H), full), pl.BlockSpec((H, H), full),
                  pl.BlockSpec((1, H), full), pl.BlockSpec((1, H), full),
                  pl.BlockSpec((1, H), full)],
        out_specs=pl.BlockSpec((bm, H), row),
        out_shape=jax.ShapeDtypeStruct((n, H), jnp.float32),
    )(d, agg, w1d, w1a, p['b1'].reshape(1, H), p['w2'],
      p['b2'].reshape(1, H), p['g'].reshape(1, H), p['be'].reshape(1, H))


NB = 3


def _sc_gather_add(ps, pd, src_idx, dst_idx, epad):
    per_w = epad // NW
    nch = per_w // CHUNK
    assert nch % NB == 0 and nch >= NB
    mesh = plsc.VectorSubcoreMesh(core_axis_name="c", subcore_axis_name="s")

    @functools.partial(
        pl.kernel, mesh=mesh,
        out_type=jax.ShapeDtypeStruct((epad, H), jnp.float32),
        scratch_types=[pltpu.VMEM((NB, CHUNK), jnp.int32),
                       pltpu.VMEM((NB, CHUNK), jnp.int32),
                       pltpu.VMEM((NB, CHUNK, H), jnp.float32)]
                      + [pltpu.SemaphoreType.DMA] * (4 * NB))
    def k(ps_hbm, pd_hbm, si_hbm, di_hbm, o_hbm,
          i1, i2, r1, *sems):
        isem = sems[0:NB]
        g1sem = sems[NB:2 * NB]
        g2sem = sems[2 * NB:3 * NB]
        wsem = sems[3 * NB:4 * NB]
        wid = lax.axis_index("s") * NC + lax.axis_index("c")
        base0 = wid * per_w

        def fire_idx(j, b):
            base = base0 + j * CHUNK
            pltpu.async_copy(si_hbm.at[pl.ds(base, CHUNK)], i1.at[b],
                             isem[b])
            pltpu.async_copy(di_hbm.at[pl.ds(base, CHUNK)], i2.at[b],
                             isem[b])

        def fire_g1(b):
            pltpu.make_async_copy(si_hbm.at[pl.ds(0, CHUNK)], i1.at[b],
                                  isem[b]).wait()
            pltpu.make_async_copy(di_hbm.at[pl.ds(0, CHUNK)], i2.at[b],
                                  isem[b]).wait()
            pltpu.async_copy(ps_hbm.at[i1.at[b]], r1.at[b], g1sem[b])

        def fire_g2(b):
            pltpu.make_async_copy(ps_hbm.at[pl.ds(0, CHUNK)], r1.at[b],
                                  g1sem[b]).wait()
            pltpu.async_copy(pd_hbm.at[i2.at[b]], r1.at[b], g2sem[b],
                             add=True)

        def drain_wb(b):
            pltpu.make_async_copy(r1.at[b], o_hbm.at[pl.ds(0, CHUNK)],
                                  wsem[b]).wait()

        fire_idx(0, 0)
        fire_idx(1, 1)
        fire_idx(2, 2)
        fire_g1(0)
        fire_g2(0)
        fire_g1(1)

        def group(jj, carry):
            for b in range(NB):
                j = jj * NB + b
                bn1 = (b + 1) % NB
                bn2 = (b + 2) % NB

                pltpu.make_async_copy(pd_hbm.at[pl.ds(0, CHUNK)], r1.at[b],
                                      g2sem[b]).wait()
                base = base0 + j * CHUNK
                pltpu.async_copy(r1.at[b], o_hbm.at[pl.ds(base, CHUNK)],
                                 wsem[b])

                @pl.when(j + 3 < nch)
                def _():
                    fire_idx(j + 3, b)

                @pl.when(j + 2 < nch)
                def _():
                    @pl.when(j + 2 >= NB)
                    def _():
                        drain_wb(bn2)
                    fire_g1(bn2)

                @pl.when(j + 1 < nch)
                def _():
                    fire_g2(bn1)
            return carry

        lax.fori_loop(0, nch // NB, group, 0)
        for b in range(NB):
            drain_wb(b)

    return k(ps, pd, src_idx, dst_idx)


def _sc_segsum_dual(vals, dst_idx, nd, epad, zeros):
    ndb = (nd + 127) // 128 * 128 + 128
    CCH = 128
    NBL = 2
    per_w = epad // NW
    nch = per_w // CCH
    rpt_z = ndb // NS
    rpt_o = (nd // NS + 7) // 8 * 8
    rpt_last = nd - (NS - 1) * rpt_o
    assert rpt_last > 0 and rpt_o % 8 == 0 and rpt_last % 8 == 0
    mesh = plsc.VectorSubcoreMesh(core_axis_name="c", subcore_axis_name="s")

    assert nch >= 2

    @functools.partial(
        pl.kernel, mesh=mesh,
        out_type=jax.ShapeDtypeStruct((NC, nd, H), jnp.float32),
        scratch_types=[pltpu.VMEM((NBL, CCH), jnp.int32),
                       pltpu.VMEM((NBL, CCH, H), jnp.float32),
                       pltpu.VMEM_SHARED((ndb, H), jnp.float32)]
                      + [pltpu.SemaphoreType.DMA] * NBL)
    def k(v_hbm, di_hbm, z_hbm, out_hbm, idx, buf, shared, *lsem):
        c = lax.axis_index("c")
        s = lax.axis_index("s")
        wid = s * NC + c
        base0 = wid * per_w

        def fire_load(j, b):
            base = base0 + j * CCH
            pltpu.async_copy(di_hbm.at[pl.ds(base, CCH)], idx.at[b],
                             lsem[b])
            pltpu.async_copy(v_hbm.at[pl.ds(base, CCH)], buf.at[b],
                             lsem[b])

        def consume(b):
            pltpu.make_async_copy(di_hbm.at[pl.ds(0, CCH)], idx.at[b],
                                  lsem[b]).wait()
            pltpu.make_async_copy(v_hbm.at[pl.ds(0, CCH)], buf.at[b],
                                  lsem[b]).wait()
            pltpu.sync_copy(buf.at[b], shared.at[idx.at[b]], add=True)

        pltpu.sync_copy(z_hbm.at[pl.ds(s * rpt_z, rpt_z)],
                        shared.at[pl.ds(s * rpt_z, rpt_z)])
        plsc.subcore_barrier()
        fire_load(0, 0)

        def group(jj, carry):
            for b in range(NBL):
                j = jj * NBL + b

                @pl.when(j + 1 < nch)
                def _():
                    fire_load(j + 1, (b + 1) % NBL)

                consume(b)
            return carry

        lax.fori_loop(0, nch // NBL, group, 0)
        if nch % NBL:
            consume((nch - 1) % NBL)
        plsc.subcore_barrier()

        @pl.when(s < NS - 1)
        def _():
            pltpu.sync_copy(shared.at[pl.ds(s * rpt_o, rpt_o)],
                            out_hbm.at[c, pl.ds(s * rpt_o, rpt_o)])

        @pl.when(s == NS - 1)
        def _():
            pltpu.sync_copy(shared.at[pl.ds((NS - 1) * rpt_o, rpt_last)],
                            out_hbm.at[c, pl.ds((NS - 1) * rpt_o, rpt_last)])

    return k(vals, dst_idx, zeros)


def _sc_segsum_chunked(vals, dst_idx, epad, zeros, csize, chunks_per_sc):
    cbuf = csize + 128
    CCH = 96
    NBL = 2
    per_t = epad // NS
    nch = per_t // CCH
    rpt_z = cbuf // NS
    rpt_o = csize // NS
    assert rpt_z % 8 == 0 and rpt_o % 8 == 0
    assert per_t % CCH == 0 and nch >= 2
    nout = NC * chunks_per_sc * csize
    mesh = plsc.VectorSubcoreMesh(core_axis_name="c", subcore_axis_name="s")

    @functools.partial(
        pl.kernel, mesh=mesh,
        out_type=jax.ShapeDtypeStruct((nout, H), jnp.float32),
        scratch_types=[pltpu.VMEM((NBL, CCH), jnp.int32),
                       pltpu.VMEM((NBL, CCH), jnp.int32),
                       pltpu.VMEM((NBL, CCH, H), jnp.float32),
                       pltpu.VMEM_SHARED((cbuf, H), jnp.float32)]
                      + [pltpu.SemaphoreType.DMA] * NBL)
    def k(v_hbm, di_hbm, z_hbm, out_hbm, idx, idx2, buf, shared, *lsem):
        c = lax.axis_index("c")
        s = lax.axis_index("s")
        base0 = s * per_t

        def fire_load(j, b):
            base = base0 + j * CCH
            pltpu.async_copy(di_hbm.at[pl.ds(base, CCH)], idx.at[b],
                             lsem[b])
            pltpu.async_copy(v_hbm.at[pl.ds(base, CCH)], buf.at[b],
                             lsem[b])

        def chunk_loop(ci, carry):
            cbase = (c * chunks_per_sc + ci) * csize

            def consume(b):
                pltpu.make_async_copy(di_hbm.at[pl.ds(0, CCH)], idx.at[b],
                                      lsem[b]).wait()
                pltpu.make_async_copy(v_hbm.at[pl.ds(0, CCH)], buf.at[b],
                                      lsem[b]).wait()
                lane = lax.broadcasted_iota(jnp.int32, (16,), 0)
                for r in range(CCH // 16):
                    v = idx[b, pl.ds(r * 16, 16)] - cbase
                    ok = (v >= 0) & (v < csize)
                    trash = csize + (r % 8) * 16 + lane
                    idx2[b, pl.ds(r * 16, 16)] = jnp.where(ok, v, trash)
                pltpu.sync_copy(buf.at[b], shared.at[idx2.at[b]],
                                add=True)

            pltpu.sync_copy(z_hbm.at[pl.ds(s * rpt_z, rpt_z)],
                            shared.at[pl.ds(s * rpt_z, rpt_z)])
            fire_load(0, 0)
            plsc.subcore_barrier()

            def group(jj, carry2):
                for b in range(NBL):
                    j = jj * NBL + b

                    @pl.when(j + 1 < nch)
                    def _():
                        fire_load(j + 1, (b + 1) % NBL)

                    consume(b)
                return carry2

            lax.fori_loop(0, nch // NBL, group, 0)
            if nch % NBL:
                consume((nch - 1) % NBL)
            plsc.subcore_barrier()
            pltpu.sync_copy(shared.at[pl.ds(s * rpt_o, rpt_o)],
                            out_hbm.at[pl.ds(cbase + s * rpt_o, rpt_o)])
            plsc.subcore_barrier()
            return carry

        lax.fori_loop(0, chunks_per_sc, chunk_loop, 0)

    return k(vals, dst_idx, zeros)


def _pad_edges(e):
    step = NW * CHUNK * NB
    return (e + step - 1) // step * step


def _interaction_mesh_dst(src_feat, dst_feat, src_idx, dst_idx, dst_trash,
                          efeat, p, nd, epad, zeros, fused_psd=False):
    w1 = p['edge']['w1']
    if fused_psd:
        psd = _matmul_bias(src_feat, jnp.concatenate([w1[:H], w1[H:2 * H]],
                                                     axis=1),
                           jnp.zeros((2 * H,), jnp.float32))
        ps, pd = psd[:, :H], psd[:, H:]
    else:
        zb = jnp.zeros((H,), jnp.float32)
        ps = _matmul_bias(src_feat, w1[:H], zb)
        pd = _matmul_bias(dst_feat, w1[H:2 * H], zb)
    g = _sc_gather_add(ps, pd, src_idx, dst_idx, epad)
    e_new = _edge_update(g, efeat, p['edge'])
    agg = _sc_segsum_dual(e_new, dst_trash, nd, epad, zeros)
    d_new = _node_update(dst_feat, agg, p['node'], dual=True)
    return d_new, e_new


def kernel(x, mesh_x, g2m_x_s, g2m_edge_index, g2m_edge_attr,
           mesh_edge_index, mesh_edge_attr, m2g_edge_index, m2g_edge_attr,
           params):
    n_grid = x.shape[1]
    n_mesh = mesh_x.shape[0]

    def prep(ei, ea, n_dst_trash):
        e = ei.shape[1]
        ep = _pad_edges(e)
        src = jnp.concatenate([ei[0], jnp.zeros((ep - e,), jnp.int32)])
        dst = jnp.concatenate([ei[1], jnp.zeros((ep - e,), jnp.int32)])
        dtr = jnp.concatenate(
            [ei[1], jnp.full((ep - e,), n_dst_trash, jnp.int32)])
        eap = jnp.concatenate(
            [ea, jnp.zeros((ep - e, ea.shape[1]), jnp.float32)])
        return src, dst, dtr, eap, ep

    g2m_src, g2m_dst, g2m_dtr, g2m_ea, ep_g2m = prep(
        g2m_edge_index, g2m_edge_attr, n_mesh)
    mm_src, mm_dst, mm_dtr, mm_ea, ep_mm = prep(
        mesh_edge_index, mesh_edge_attr, n_mesh)
    m2g_src, m2g_dst, m2g_dtr, m2g_ea, ep_m2g = prep(
        m2g_edge_index, m2g_edge_attr, 1 << 29)

    zeros_mesh = jnp.zeros((n_mesh + 16, H), jnp.float32)
    csize = 12544
    zeros_grid = jnp.zeros((csize + 128, H), jnp.float32)

    grid_in = jnp.concatenate([jnp.squeeze(x, 0), g2m_x_s], axis=-1)
    grid = _mlp(grid_in, params['grid_embed'])
    pme = params['mesh_embed']
    mesh = _mlp(mesh_x, {'w1': pme['w1'][-mesh_x.shape[1]:], 'b1': pme['b1'],
                         'w2': pme['w2'], 'b2': pme['b2'], 'g': pme['g'],
                         'be': pme['be']})
    e = _mlp(g2m_ea, params['g2m_edge_embed'])
    mesh, _ = _interaction_mesh_dst(
        grid, mesh, g2m_src, g2m_dst, g2m_dtr, e, params['g2m_gnn'],
        n_mesh, ep_g2m, zeros_mesh)
    grid = _mlp(grid, params['grid_model'], residual=True)

    em = _mlp(mm_ea, params['proc_edge_embed'])
    for p in params['proc']:
        mesh, em = _interaction_mesh_dst(
            mesh, mesh, mm_src, mm_dst, mm_dtr, em, p,
            n_mesh, ep_mm, zeros_mesh, fused_psd=True)

    eg = _mlp(m2g_ea, params['m2g_edge_embed'])
    pg = params['m2g_gnn']
    w1 = pg['edge']['w1']
    zb = jnp.zeros((H,), jnp.float32)
    ps = _matmul_bias(mesh, w1[:H], zb)
    pd = _matmul_bias(grid, w1[H:2 * H], zb)
    g = _sc_gather_add(ps, pd, m2g_src, m2g_dst, ep_m2g)
    e_new = _edge_update(g, eg, pg['edge'])
    agg = _sc_segsum_chunked(e_new, m2g_dtr, ep_m2g, zeros_grid,
                             csize, chunks_per_sc=2)
    grid = _node_update(grid, agg[:n_grid], pg['node'], dual=False)

    out = _mlp(grid, params['final'], norm=False)
    return out[None]
